# Initial kernel scaffold; baseline (speedup 1.0000x reference)
import jax
import jax.numpy as jnp
from jax.experimental import pallas as pl


def kernel(x, edge_index, W1, b1, W2, b2):
    raise NotImplementedError("write your pallas kernel here")


# speedup vs baseline: 9.1761x; 9.1761x over previous
import functools

import jax
import jax.numpy as jnp
from jax import lax
from jax.experimental import pallas as pl
from jax.experimental.pallas import tpu as pltpu
from jax.experimental.pallas import tpu_sc as plsc

N_NODES = 10000
N_PAD = 10240
IN_DIM = 128
HID_DIM = 128
OUT_DIM = 64
N_EDGES = 320000
EDGE_BLOCK = 128
N_WORKERS = 32
BLOCKS_PER_W = 80
N_BLOCKS = N_WORKERS * BLOCKS_PER_W
E_PAD = N_BLOCKS * EDGE_BLOCK
ROWS_PER_TILE = N_PAD // 16

_MESH = plsc.VectorSubcoreMesh(core_axis_name="c", subcore_axis_name="s")


def _zero_vmem_2d(ref, nrows, d):
    z = jnp.zeros((16,), jnp.float32)

    def body(r, carry):
        for k in range(d // 16):
            ref[r, pl.ds(k * 16, 16)] = z
        return carry

    lax.fori_loop(0, nrows, body, 0)


def _deg_kernel_body(dstb_hbm, out_hbm, dst_slab, ones_v, stage, acc, sem):
    c = lax.axis_index("c")
    s = lax.axis_index("s")
    wid = c * 16 + s

    pltpu.sync_copy(dstb_hbm.at[pl.ds(wid * BLOCKS_PER_W, BLOCKS_PER_W)], dst_slab)

    one = jnp.ones((16,), jnp.float32)
    zero = jnp.zeros((16,), jnp.float32)
    for k in range(EDGE_BLOCK // 16):
        ones_v[pl.ds(k * 16, 16)] = one

    def zbody(r, carry):
        stage[pl.ds(r * 16, 16)] = zero
        return carry

    lax.fori_loop(0, ROWS_PER_TILE // 16, zbody, 0)
    pltpu.sync_copy(stage, acc.at[pl.ds(s * ROWS_PER_TILE, ROWS_PER_TILE)])
    plsc.subcore_barrier()

    def ebody(j, carry):
        pltpu.sync_copy(ones_v, acc.at[dst_slab.at[j]], add=True)
        return carry

    lax.fori_loop(0, BLOCKS_PER_W, ebody, 0)
    plsc.subcore_barrier()

    pltpu.sync_copy(acc.at[pl.ds(s * ROWS_PER_TILE, ROWS_PER_TILE)], stage)
    pltpu.sync_copy(
        stage, out_hbm.at[pl.ds(c * N_PAD + s * ROWS_PER_TILE, ROWS_PER_TILE)])


def _sc_degree(dst_blocks):
    return pl.kernel(
        _deg_kernel_body,
        out_type=jax.ShapeDtypeStruct((2 * N_PAD,), jnp.float32),
        mesh=_MESH,
        scratch_types=[
            pltpu.VMEM((BLOCKS_PER_W, EDGE_BLOCK), jnp.int32),
            pltpu.VMEM((EDGE_BLOCK,), jnp.float32),
            pltpu.VMEM((ROWS_PER_TILE,), jnp.float32),
            pltpu.VMEM_SHARED((N_PAD,), jnp.float32),
            pltpu.SemaphoreType.DMA,
        ],
    )(dst_blocks)


def _agg_kernel_body(d, g_hbm, srcb_hbm, dstb_hbm, out_hbm,
                     src_slab, dst_slab, rows_v, acc, sem):
    stage = rows_v
    c = lax.axis_index("c")
    s = lax.axis_index("s")
    wid = c * 16 + s

    pltpu.sync_copy(srcb_hbm.at[pl.ds(wid * BLOCKS_PER_W, BLOCKS_PER_W)], src_slab)
    pltpu.sync_copy(dstb_hbm.at[pl.ds(wid * BLOCKS_PER_W, BLOCKS_PER_W)], dst_slab)

    _zero_vmem_2d(stage, EDGE_BLOCK, d)
    for t in range(ROWS_PER_TILE // EDGE_BLOCK):
        pltpu.sync_copy(
            stage, acc.at[pl.ds(s * ROWS_PER_TILE + t * EDGE_BLOCK, EDGE_BLOCK)])
    plsc.subcore_barrier()

    def ebody(j, carry):
        pltpu.async_copy(g_hbm.at[src_slab.at[j]], rows_v, sem).wait()
        pltpu.sync_copy(rows_v, acc.at[dst_slab.at[j]], add=True)
        return carry

    lax.fori_loop(0, BLOCKS_PER_W, ebody, 0)
    plsc.subcore_barrier()

    for t in range(ROWS_PER_TILE // EDGE_BLOCK):
        r = s * ROWS_PER_TILE + t * EDGE_BLOCK
        pltpu.sync_copy(acc.at[pl.ds(r, EDGE_BLOCK)], stage)
        pltpu.sync_copy(stage, out_hbm.at[c, pl.ds(r, EDGE_BLOCK)])


def _sc_aggregate(g, src_blocks, dst_blocks, d):
    return pl.kernel(
        functools.partial(_agg_kernel_body, d),
        out_type=jax.ShapeDtypeStruct((2, N_PAD, d), jnp.float32),
        mesh=_MESH,
        scratch_types=[
            pltpu.VMEM((BLOCKS_PER_W, EDGE_BLOCK), jnp.int32),
            pltpu.VMEM((BLOCKS_PER_W, EDGE_BLOCK), jnp.int32),
            pltpu.VMEM((EDGE_BLOCK, d), jnp.float32),
            pltpu.VMEM_SHARED((N_PAD, d), jnp.float32),
            pltpu.SemaphoreType.DMA,
        ],
    )(g, src_blocks, dst_blocks)


_TC_GRID = 8
_TC_ROWS = N_PAD // _TC_GRID


def _tc_b_body(d0, d1, x, w1, g1):
    dinv = lax.rsqrt(d0[...] + d1[...] + 1.0)
    h = jnp.dot(x[...], w1[...], preferred_element_type=jnp.float32)
    g1[...] = dinv * h


def _tc_d_body(d0, d1, a0, a1, g1, b1, w2, g2):
    dinv = lax.rsqrt(d0[...] + d1[...] + 1.0)
    out1 = dinv * (a0[...] + a1[...] + g1[...]) + b1[...]
    h = jnp.maximum(out1, 0.0)
    mm = dinv * jnp.dot(h, w2[...], preferred_element_type=jnp.float32)
    g2[...] = jnp.concatenate(
        [mm, jnp.zeros((mm.shape[0], HID_DIM - OUT_DIM), jnp.float32)], axis=1)


def _tc_f_body(d0, d1, a0, a1, g2, b2, y):
    dinv = lax.rsqrt(d0[...] + d1[...] + 1.0)
    o = (dinv * (a0[...] + a1[...] + g2[...]))[:, :OUT_DIM] + b2[...]
    m = jnp.max(o, axis=1, keepdims=True)
    e = jnp.exp(o - m)
    y[...] = (o - m) - jnp.log(jnp.sum(e, axis=1, keepdims=True))


def _col_spec():
    return pl.BlockSpec((_TC_ROWS, 1), lambda i: (i, 0))


def _mat_spec(d):
    return pl.BlockSpec((_TC_ROWS, d), lambda i: (i, 0))


def _full_spec(r, c):
    return pl.BlockSpec((r, c), lambda i: (0, 0))


def kernel(x, edge_index, W1, b1, W2, b2):
    x = x.astype(jnp.float32)
    src = edge_index[0].astype(jnp.int32)
    dst = edge_index[1].astype(jnp.int32)

    n_fill = E_PAD - N_EDGES
    src_blocks = jnp.concatenate(
        [src, jnp.zeros((n_fill,), jnp.int32)]).reshape(N_BLOCKS, EDGE_BLOCK)
    dst_blocks = jnp.concatenate(
        [dst, jnp.full((n_fill,), N_PAD - 1, jnp.int32)]).reshape(N_BLOCKS, EDGE_BLOCK)

    x_pad = jnp.concatenate(
        [x, jnp.zeros((N_PAD - N_NODES, IN_DIM), jnp.float32)])

    deg_p = _sc_degree(dst_blocks).reshape(2, N_PAD)
    d0 = deg_p[0].reshape(N_PAD, 1)
    d1 = deg_p[1].reshape(N_PAD, 1)

    g1 = pl.pallas_call(
        _tc_b_body,
        grid=(_TC_GRID,),
        in_specs=[_col_spec(), _col_spec(), _mat_spec(IN_DIM),
                  _full_spec(IN_DIM, HID_DIM)],
        out_specs=_mat_spec(HID_DIM),
        out_shape=jax.ShapeDtypeStruct((N_PAD, HID_DIM), jnp.float32),
    )(d0, d1, x_pad, W1)

    agg1 = _sc_aggregate(g1, src_blocks, dst_blocks, HID_DIM)

    g2 = pl.pallas_call(
        _tc_d_body,
        grid=(_TC_GRID,),
        in_specs=[_col_spec(), _col_spec(), _mat_spec(HID_DIM),
                  _mat_spec(HID_DIM), _mat_spec(HID_DIM),
                  _full_spec(1, HID_DIM), _full_spec(HID_DIM, OUT_DIM)],
        out_specs=_mat_spec(HID_DIM),
        out_shape=jax.ShapeDtypeStruct((N_PAD, HID_DIM), jnp.float32),
    )(d0, d1, agg1[0], agg1[1], g1, b1.reshape(1, HID_DIM), W2)

    agg2 = _sc_aggregate(g2, src_blocks, dst_blocks, HID_DIM)

    y = pl.pallas_call(
        _tc_f_body,
        grid=(_TC_GRID,),
        in_specs=[_col_spec(), _col_spec(), _mat_spec(HID_DIM),
                  _mat_spec(HID_DIM), _mat_spec(HID_DIM),
                  _full_spec(1, OUT_DIM)],
        out_specs=_mat_spec(OUT_DIM),
        out_shape=jax.ShapeDtypeStruct((N_PAD, OUT_DIM), jnp.float32),
    )(d0, d1, agg2[0], agg2[1], g2, b2.reshape(1, OUT_DIM))

    return y[:N_NODES]


# speedup vs baseline: 9.6739x; 1.0542x over previous
import functools

import jax
import jax.numpy as jnp
from jax import lax
from jax.experimental import pallas as pl
from jax.experimental.pallas import tpu as pltpu
from jax.experimental.pallas import tpu_sc as plsc

N_NODES = 10000
N_PAD = 10240
IN_DIM = 128
HID_DIM = 128
OUT_DIM = 64
N_EDGES = 320000
EDGE_BLOCK = 128
N_WORKERS = 32
BLOCKS_PER_W = 80
N_BLOCKS = N_WORKERS * BLOCKS_PER_W
IDX_CHUNK = 16
E_PAD = N_BLOCKS * EDGE_BLOCK
ROWS_PER_TILE = N_PAD // 16

_MESH = plsc.VectorSubcoreMesh(core_axis_name="c", subcore_axis_name="s")


def _zero_vmem_2d(ref, nrows, d):
    z = jnp.zeros((16,), jnp.float32)

    def body(r, carry):
        for k in range(d // 16):
            ref[r, pl.ds(k * 16, 16)] = z
        return carry

    lax.fori_loop(0, nrows, body, 0)


def _deg_kernel_body(dstb_hbm, out_hbm, dst_slab, ones_v, stage, acc, sem):
    c = lax.axis_index("c")
    s = lax.axis_index("s")
    wid = c * 16 + s

    pltpu.sync_copy(dstb_hbm.at[pl.ds(wid * BLOCKS_PER_W, BLOCKS_PER_W)], dst_slab)

    one = jnp.ones((16,), jnp.float32)
    zero = jnp.zeros((16,), jnp.float32)
    for k in range(EDGE_BLOCK // 16):
        ones_v[pl.ds(k * 16, 16)] = one

    def zbody(r, carry):
        stage[pl.ds(r * 16, 16)] = zero
        return carry

    lax.fori_loop(0, ROWS_PER_TILE // 16, zbody, 0)
    pltpu.sync_copy(stage, acc.at[pl.ds(s * ROWS_PER_TILE, ROWS_PER_TILE)])
    plsc.subcore_barrier()

    def ebody(j, carry):
        pltpu.sync_copy(ones_v, acc.at[dst_slab.at[j]], add=True)
        return carry

    lax.fori_loop(0, BLOCKS_PER_W, ebody, 0)
    plsc.subcore_barrier()

    pltpu.sync_copy(acc.at[pl.ds(s * ROWS_PER_TILE, ROWS_PER_TILE)], stage)
    pltpu.sync_copy(
        stage, out_hbm.at[pl.ds(c * N_PAD + s * ROWS_PER_TILE, ROWS_PER_TILE)])


def _sc_degree(dst_blocks):
    return pl.kernel(
        _deg_kernel_body,
        out_type=jax.ShapeDtypeStruct((2 * N_PAD,), jnp.float32),
        mesh=_MESH,
        scratch_types=[
            pltpu.VMEM((BLOCKS_PER_W, EDGE_BLOCK), jnp.int32),
            pltpu.VMEM((EDGE_BLOCK,), jnp.float32),
            pltpu.VMEM((ROWS_PER_TILE,), jnp.float32),
            pltpu.VMEM_SHARED((N_PAD,), jnp.float32),
            pltpu.SemaphoreType.DMA,
        ],
    )(dst_blocks)


def _agg_kernel_body(d, g_hbm, srcb_hbm, dstb_hbm, out_hbm,
                     src_slab, dst_slab, buf0, buf1, acc, sem0, sem1):
    stage = buf0
    c = lax.axis_index("c")
    s = lax.axis_index("s")
    wid = c * 16 + s

    _zero_vmem_2d(stage, EDGE_BLOCK, d)
    for t in range(ROWS_PER_TILE // EDGE_BLOCK):
        pltpu.sync_copy(
            stage, acc.at[pl.ds(s * ROWS_PER_TILE + t * EDGE_BLOCK, EDGE_BLOCK)])
    plsc.subcore_barrier()

    def chunk_body(ci, carry):
        base = wid * BLOCKS_PER_W + ci * IDX_CHUNK
        pltpu.sync_copy(srcb_hbm.at[pl.ds(base, IDX_CHUNK)], src_slab)
        pltpu.sync_copy(dstb_hbm.at[pl.ds(base, IDX_CHUNK)], dst_slab)

        pltpu.async_copy(g_hbm.at[src_slab.at[0]], buf0, sem0)

        def ebody2(i, c2):
            j0 = 2 * i
            j1 = j0 + 1
            pltpu.async_copy(g_hbm.at[src_slab.at[j1]], buf1, sem1)
            pltpu.make_async_copy(g_hbm.at[src_slab.at[j0]], buf0, sem0).wait()
            pltpu.sync_copy(buf0, acc.at[dst_slab.at[j0]], add=True)
            jn = jnp.minimum(j0 + 2, IDX_CHUNK - 1)
            pltpu.async_copy(g_hbm.at[src_slab.at[jn]], buf0, sem0)
            pltpu.make_async_copy(g_hbm.at[src_slab.at[j1]], buf1, sem1).wait()
            pltpu.sync_copy(buf1, acc.at[dst_slab.at[j1]], add=True)
            return c2

        lax.fori_loop(0, IDX_CHUNK // 2, ebody2, 0)
        pltpu.make_async_copy(g_hbm.at[src_slab.at[0]], buf0, sem0).wait()
        return carry

    lax.fori_loop(0, BLOCKS_PER_W // IDX_CHUNK, chunk_body, 0)
    plsc.subcore_barrier()

    for t in range(ROWS_PER_TILE // EDGE_BLOCK):
        r = s * ROWS_PER_TILE + t * EDGE_BLOCK
        pltpu.sync_copy(acc.at[pl.ds(r, EDGE_BLOCK)], stage)
        pltpu.sync_copy(stage, out_hbm.at[c, pl.ds(r, EDGE_BLOCK)])


def _sc_aggregate(g, src_blocks, dst_blocks, d):
    return pl.kernel(
        functools.partial(_agg_kernel_body, d),
        out_type=jax.ShapeDtypeStruct((2, N_PAD, d), jnp.float32),
        mesh=_MESH,
        scratch_types=[
            pltpu.VMEM((IDX_CHUNK, EDGE_BLOCK), jnp.int32),
            pltpu.VMEM((IDX_CHUNK, EDGE_BLOCK), jnp.int32),
            pltpu.VMEM((EDGE_BLOCK, d), jnp.float32),
            pltpu.VMEM((EDGE_BLOCK, d), jnp.float32),
            pltpu.VMEM_SHARED((N_PAD, d), jnp.float32),
            pltpu.SemaphoreType.DMA,
            pltpu.SemaphoreType.DMA,
        ],
    )(g, src_blocks, dst_blocks)


_TC_GRID = 8
_TC_ROWS = N_PAD // _TC_GRID


def _tc_b_body(d0, d1, x, w1, g1):
    dinv = lax.rsqrt(d0[...] + d1[...] + 1.0)
    h = jnp.dot(x[...], w1[...], preferred_element_type=jnp.float32)
    g1[...] = dinv * h


def _tc_d_body(d0, d1, a0, a1, g1, b1, w2, g2):
    dinv = lax.rsqrt(d0[...] + d1[...] + 1.0)
    out1 = dinv * (a0[...] + a1[...] + g1[...]) + b1[...]
    h = jnp.maximum(out1, 0.0)
    mm = dinv * jnp.dot(h, w2[...], preferred_element_type=jnp.float32)
    g2[...] = jnp.concatenate(
        [mm, jnp.zeros((mm.shape[0], HID_DIM - OUT_DIM), jnp.float32)], axis=1)


def _tc_f_body(d0, d1, a0, a1, g2, b2, y):
    dinv = lax.rsqrt(d0[...] + d1[...] + 1.0)
    o = (dinv * (a0[...] + a1[...] + g2[...]))[:, :OUT_DIM] + b2[...]
    m = jnp.max(o, axis=1, keepdims=True)
    e = jnp.exp(o - m)
    y[...] = (o - m) - jnp.log(jnp.sum(e, axis=1, keepdims=True))


def _col_spec():
    return pl.BlockSpec((_TC_ROWS, 1), lambda i: (i, 0))


def _mat_spec(d):
    return pl.BlockSpec((_TC_ROWS, d), lambda i: (i, 0))


def _full_spec(r, c):
    return pl.BlockSpec((r, c), lambda i: (0, 0))


def kernel(x, edge_index, W1, b1, W2, b2):
    x = x.astype(jnp.float32)
    src = edge_index[0].astype(jnp.int32)
    dst = edge_index[1].astype(jnp.int32)

    n_fill = E_PAD - N_EDGES
    src_blocks = jnp.concatenate(
        [src, jnp.zeros((n_fill,), jnp.int32)]).reshape(N_BLOCKS, EDGE_BLOCK)
    dst_blocks = jnp.concatenate(
        [dst, jnp.full((n_fill,), N_PAD - 1, jnp.int32)]).reshape(N_BLOCKS, EDGE_BLOCK)

    x_pad = jnp.concatenate(
        [x, jnp.zeros((N_PAD - N_NODES, IN_DIM), jnp.float32)])

    deg_p = _sc_degree(dst_blocks).reshape(2, N_PAD)
    d0 = deg_p[0].reshape(N_PAD, 1)
    d1 = deg_p[1].reshape(N_PAD, 1)

    g1 = pl.pallas_call(
        _tc_b_body,
        grid=(_TC_GRID,),
        in_specs=[_col_spec(), _col_spec(), _mat_spec(IN_DIM),
                  _full_spec(IN_DIM, HID_DIM)],
        out_specs=_mat_spec(HID_DIM),
        out_shape=jax.ShapeDtypeStruct((N_PAD, HID_DIM), jnp.float32),
    )(d0, d1, x_pad, W1)

    agg1 = _sc_aggregate(g1, src_blocks, dst_blocks, HID_DIM)

    g2 = pl.pallas_call(
        _tc_d_body,
        grid=(_TC_GRID,),
        in_specs=[_col_spec(), _col_spec(), _mat_spec(HID_DIM),
                  _mat_spec(HID_DIM), _mat_spec(HID_DIM),
                  _full_spec(1, HID_DIM), _full_spec(HID_DIM, OUT_DIM)],
        out_specs=_mat_spec(HID_DIM),
        out_shape=jax.ShapeDtypeStruct((N_PAD, HID_DIM), jnp.float32),
    )(d0, d1, agg1[0], agg1[1], g1, b1.reshape(1, HID_DIM), W2)

    agg2 = _sc_aggregate(g2, src_blocks, dst_blocks, HID_DIM)

    y = pl.pallas_call(
        _tc_f_body,
        grid=(_TC_GRID,),
        in_specs=[_col_spec(), _col_spec(), _mat_spec(HID_DIM),
                  _mat_spec(HID_DIM), _mat_spec(HID_DIM),
                  _full_spec(1, OUT_DIM)],
        out_specs=_mat_spec(OUT_DIM),
        out_shape=jax.ShapeDtypeStruct((N_PAD, OUT_DIM), jnp.float32),
    )(d0, d1, agg2[0], agg2[1], g2, b2.reshape(1, OUT_DIM))

    return y[:N_NODES]


# speedup vs baseline: 9.6756x; 1.0002x over previous
import functools

import jax
import jax.numpy as jnp
from jax import lax
from jax.experimental import pallas as pl
from jax.experimental.pallas import tpu as pltpu
from jax.experimental.pallas import tpu_sc as plsc

N_NODES = 10000
N_PAD = 10240
IN_DIM = 128
HID_DIM = 128
OUT_DIM = 64
N_EDGES = 320000
EDGE_BLOCK = 128
N_WORKERS = 32
BLOCKS_PER_W = 80
N_BLOCKS = N_WORKERS * BLOCKS_PER_W
IDX_CHUNK = 16
E_PAD = N_BLOCKS * EDGE_BLOCK
ROWS_PER_TILE = N_PAD // 16

_MESH = plsc.VectorSubcoreMesh(core_axis_name="c", subcore_axis_name="s")


def _zero_vmem_2d(ref, nrows, d):
    z = jnp.zeros((16,), jnp.float32)

    def body(r, carry):
        for k in range(d // 16):
            ref[r, pl.ds(k * 16, 16)] = z
        return carry

    lax.fori_loop(0, nrows, body, 0)


def _deg_kernel_body(dstb_hbm, out_hbm, dst_slab, ones_v, stage, acc, sem):
    c = lax.axis_index("c")
    s = lax.axis_index("s")
    wid = c * 16 + s

    pltpu.sync_copy(dstb_hbm.at[pl.ds(wid * BLOCKS_PER_W, BLOCKS_PER_W)], dst_slab)

    one = jnp.ones((16,), jnp.float32)
    zero = jnp.zeros((16,), jnp.float32)
    for k in range(EDGE_BLOCK // 16):
        ones_v[pl.ds(k * 16, 16)] = one

    def zbody(r, carry):
        stage[pl.ds(r * 16, 16)] = zero
        return carry

    lax.fori_loop(0, ROWS_PER_TILE // 16, zbody, 0)
    pltpu.sync_copy(stage, acc.at[pl.ds(s * ROWS_PER_TILE, ROWS_PER_TILE)])
    plsc.subcore_barrier()

    def ebody(j, carry):
        pltpu.sync_copy(ones_v, acc.at[dst_slab.at[j]], add=True)
        return carry

    lax.fori_loop(0, BLOCKS_PER_W, ebody, 0)
    plsc.subcore_barrier()

    pltpu.sync_copy(acc.at[pl.ds(s * ROWS_PER_TILE, ROWS_PER_TILE)], stage)
    pltpu.sync_copy(
        stage, out_hbm.at[pl.ds(c * N_PAD + s * ROWS_PER_TILE, ROWS_PER_TILE)])


def _sc_degree(dst_blocks):
    return pl.kernel(
        _deg_kernel_body,
        out_type=jax.ShapeDtypeStruct((2 * N_PAD,), jnp.float32),
        mesh=_MESH,
        scratch_types=[
            pltpu.VMEM((BLOCKS_PER_W, EDGE_BLOCK), jnp.int32),
            pltpu.VMEM((EDGE_BLOCK,), jnp.float32),
            pltpu.VMEM((ROWS_PER_TILE,), jnp.float32),
            pltpu.VMEM_SHARED((N_PAD,), jnp.float32),
            pltpu.SemaphoreType.DMA,
        ],
    )(dst_blocks)


def _agg_kernel_body(d, g_hbm, srcb_hbm, dstb_hbm, out_hbm,
                     src_slab, dst_slab, buf0, buf1, acc, sem0, sem1):
    stage = buf0
    c = lax.axis_index("c")
    s = lax.axis_index("s")
    wid = c * 16 + s

    _zero_vmem_2d(stage, EDGE_BLOCK, d)
    for t in range(ROWS_PER_TILE // EDGE_BLOCK):
        pltpu.sync_copy(
            stage, acc.at[pl.ds(s * ROWS_PER_TILE + t * EDGE_BLOCK, EDGE_BLOCK)])
    plsc.subcore_barrier()

    def chunk_body(ci, carry):
        base = wid * BLOCKS_PER_W + ci * IDX_CHUNK
        pltpu.sync_copy(srcb_hbm.at[pl.ds(base, IDX_CHUNK)], src_slab)
        pltpu.sync_copy(dstb_hbm.at[pl.ds(base, IDX_CHUNK)], dst_slab)

        pltpu.async_copy(g_hbm.at[src_slab.at[0]], buf0, sem0)

        def ebody2(i, c2):
            j0 = 2 * i
            pltpu.async_copy(g_hbm.at[src_slab.at[j0 + 1]], buf1, sem1)

            pltpu.make_async_copy(g_hbm.at[src_slab.at[j0]], buf0, sem0).wait()
            pltpu.sync_copy(buf0, acc.at[dst_slab.at[j0]], add=True)
            jn = jnp.minimum(j0 + 2, IDX_CHUNK - 1)
            pltpu.async_copy(g_hbm.at[src_slab.at[jn]], buf0, sem0)

            pltpu.make_async_copy(g_hbm.at[src_slab.at[j0 + 1]], buf1, sem1).wait()
            pltpu.sync_copy(buf1, acc.at[dst_slab.at[j0 + 1]], add=True)
            return c2

        lax.fori_loop(0, IDX_CHUNK // 2, ebody2, 0)
        pltpu.make_async_copy(g_hbm.at[src_slab.at[0]], buf0, sem0).wait()
        return carry

    lax.fori_loop(0, BLOCKS_PER_W // IDX_CHUNK, chunk_body, 0)
    plsc.subcore_barrier()

    for t in range(ROWS_PER_TILE // EDGE_BLOCK):
        r = s * ROWS_PER_TILE + t * EDGE_BLOCK
        pltpu.sync_copy(acc.at[pl.ds(r, EDGE_BLOCK)], stage)
        pltpu.sync_copy(stage, out_hbm.at[c, pl.ds(r, EDGE_BLOCK)])


def _sc_aggregate(g, src_blocks, dst_blocks, d):
    return pl.kernel(
        functools.partial(_agg_kernel_body, d),
        out_type=jax.ShapeDtypeStruct((2, N_PAD, d), jnp.float32),
        mesh=_MESH,
        scratch_types=[
            pltpu.VMEM((IDX_CHUNK, EDGE_BLOCK), jnp.int32),
            pltpu.VMEM((IDX_CHUNK, EDGE_BLOCK), jnp.int32),
            pltpu.VMEM((EDGE_BLOCK, d), jnp.float32),
            pltpu.VMEM((EDGE_BLOCK, d), jnp.float32),
            pltpu.VMEM_SHARED((N_PAD, d), jnp.float32),
            pltpu.SemaphoreType.DMA,
            pltpu.SemaphoreType.DMA,
        ],
    )(g, src_blocks, dst_blocks)


_TC_GRID = 8
_TC_ROWS = N_PAD // _TC_GRID


def _tc_b_body(d0, d1, x, w1, g1):
    dinv = lax.rsqrt(d0[...] + d1[...] + 1.0)
    h = jnp.dot(x[...], w1[...], preferred_element_type=jnp.float32)
    g1[...] = dinv * h


def _tc_d_body(d0, d1, a0, a1, g1, b1, w2, g2):
    dinv = lax.rsqrt(d0[...] + d1[...] + 1.0)
    out1 = dinv * (a0[...] + a1[...] + g1[...]) + b1[...]
    h = jnp.maximum(out1, 0.0)
    mm = dinv * jnp.dot(h, w2[...], preferred_element_type=jnp.float32)
    g2[...] = jnp.concatenate(
        [mm, jnp.zeros((mm.shape[0], HID_DIM - OUT_DIM), jnp.float32)], axis=1)


def _tc_f_body(d0, d1, a0, a1, g2, b2, y):
    dinv = lax.rsqrt(d0[...] + d1[...] + 1.0)
    o = (dinv * (a0[...] + a1[...] + g2[...]))[:, :OUT_DIM] + b2[...]
    m = jnp.max(o, axis=1, keepdims=True)
    e = jnp.exp(o - m)
    y[...] = (o - m) - jnp.log(jnp.sum(e, axis=1, keepdims=True))


def _col_spec():
    return pl.BlockSpec((_TC_ROWS, 1), lambda i: (i, 0))


def _mat_spec(d):
    return pl.BlockSpec((_TC_ROWS, d), lambda i: (i, 0))


def _full_spec(r, c):
    return pl.BlockSpec((r, c), lambda i: (0, 0))


def kernel(x, edge_index, W1, b1, W2, b2):
    x = x.astype(jnp.float32)
    src = edge_index[0].astype(jnp.int32)
    dst = edge_index[1].astype(jnp.int32)

    n_fill = E_PAD - N_EDGES
    src_blocks = jnp.concatenate(
        [src, jnp.zeros((n_fill,), jnp.int32)]).reshape(N_BLOCKS, EDGE_BLOCK)
    dst_blocks = jnp.concatenate(
        [dst, jnp.full((n_fill,), N_PAD - 1, jnp.int32)]).reshape(N_BLOCKS, EDGE_BLOCK)

    x_pad = jnp.concatenate(
        [x, jnp.zeros((N_PAD - N_NODES, IN_DIM), jnp.float32)])

    deg_p = _sc_degree(dst_blocks).reshape(2, N_PAD)
    d0 = deg_p[0].reshape(N_PAD, 1)
    d1 = deg_p[1].reshape(N_PAD, 1)

    g1 = pl.pallas_call(
        _tc_b_body,
        grid=(_TC_GRID,),
        in_specs=[_col_spec(), _col_spec(), _mat_spec(IN_DIM),
                  _full_spec(IN_DIM, HID_DIM)],
        out_specs=_mat_spec(HID_DIM),
        out_shape=jax.ShapeDtypeStruct((N_PAD, HID_DIM), jnp.float32),
    )(d0, d1, x_pad, W1)

    agg1 = _sc_aggregate(g1, src_blocks, dst_blocks, HID_DIM)

    g2 = pl.pallas_call(
        _tc_d_body,
        grid=(_TC_GRID,),
        in_specs=[_col_spec(), _col_spec(), _mat_spec(HID_DIM),
                  _mat_spec(HID_DIM), _mat_spec(HID_DIM),
                  _full_spec(1, HID_DIM), _full_spec(HID_DIM, OUT_DIM)],
        out_specs=_mat_spec(HID_DIM),
        out_shape=jax.ShapeDtypeStruct((N_PAD, HID_DIM), jnp.float32),
    )(d0, d1, agg1[0], agg1[1], g1, b1.reshape(1, HID_DIM), W2)

    agg2 = _sc_aggregate(g2, src_blocks, dst_blocks, HID_DIM)

    y = pl.pallas_call(
        _tc_f_body,
        grid=(_TC_GRID,),
        in_specs=[_col_spec(), _col_spec(), _mat_spec(HID_DIM),
                  _mat_spec(HID_DIM), _mat_spec(HID_DIM),
                  _full_spec(1, OUT_DIM)],
        out_specs=_mat_spec(OUT_DIM),
        out_shape=jax.ShapeDtypeStruct((N_PAD, OUT_DIM), jnp.float32),
    )(d0, d1, agg2[0], agg2[1], g2, b2.reshape(1, OUT_DIM))

    return y[:N_NODES]
